# Initial kernel scaffold; baseline (speedup 1.0000x reference)
import jax
import jax.numpy as jnp
from jax.experimental import pallas as pl


def kernel(x, radj, inxs, Wq, Wk, Wv, Wo, g1, be1, W1, bf1, W2, bf2, g2, be2):
    raise NotImplementedError("write your pallas kernel here")


# speedup vs baseline: 22.8635x; 22.8635x over previous
import functools
import math

import jax
import jax.numpy as jnp
from jax import lax
from jax.experimental import pallas as pl
from jax.experimental.pallas import tpu as pltpu
from jax.experimental.pallas import tpu_sc as plsc

_B, _T, _C, _K = 64, 200, 128, 16
_DFF = int(_C * 1.5)
_NC, _NS, _L = 2, 16, 16
_NW = _NC * _NS
_BPW = _B // _NW
_NG = (_T + _L - 1) // _L


def _s1_body(x_ref, wq_ref, wk_ref, s_ref):
    xb = x_ref[0]
    q = jnp.dot(xb, wq_ref[...], preferred_element_type=jnp.float32)
    k = jnp.dot(xb, wk_ref[...], preferred_element_type=jnp.float32)
    s = lax.dot_general(q, k, (((1,), (1,)), ((), ())),
                        preferred_element_type=jnp.float32)
    s_ref[0] = s * (1.0 / math.sqrt(_C))


def _scores(x, Wq, Wk):
    return pl.pallas_call(
        _s1_body,
        grid=(_B,),
        in_specs=[
            pl.BlockSpec((1, _T, _C), lambda b: (b, 0, 0)),
            pl.BlockSpec((_C, _C), lambda b: (0, 0)),
            pl.BlockSpec((_C, _C), lambda b: (0, 0)),
        ],
        out_specs=pl.BlockSpec((1, _T, _T), lambda b: (b, 0, 0)),
        out_shape=jax.ShapeDtypeStruct((_B, _T, _T), jnp.float32),
    )(x, Wq, Wk)


def _sc_body(s_hbm, idx_hbm, radj_hbm, a_hbm, s_v, a_v, idx_v, radj_v,
             sbuf, ibuf):
    wid = lax.axis_index("s") * _NC + lax.axis_index("c")
    zero16 = jnp.zeros((_L,), jnp.float32)
    lanes = jnp.arange(_L, dtype=jnp.int32)

    for i in range(_BPW):
        b = wid * _BPW + i
        pltpu.sync_copy(s_hbm.at[b], s_v)
        pltpu.sync_copy(idx_hbm.at[b], idx_v)
        pltpu.sync_copy(radj_hbm.at[b], radj_v)

        def _zrow(t, _):
            for j in range(_NG):
                a_v[t, pl.ds(min(j * _L, _T - _L), _L)] = zero16
            return _
        lax.fori_loop(0, _T, _zrow, None)

        def _group(g, _):
            tb = g * _L
            rows_raw = tb + lanes
            valid = rows_raw < _T
            rows = jnp.minimum(rows_raw, _T - 1)
            m = jnp.full((_L,), -jnp.inf, jnp.float32)
            for k in range(_K):
                kk = jnp.full((_L,), k, jnp.int32)
                nk = plsc.load_gather(idx_v, [rows, kk])
                rk = plsc.load_gather(radj_v, [rows, kk])
                sk = plsc.load_gather(s_v, [rows, nk]) + rk
                m = jnp.maximum(m, sk)
                ibuf[k] = nk
                sbuf[k] = sk
            ssum = zero16
            for k in range(_K):
                e = jnp.exp(sbuf[k] - m)
                ssum = ssum + e
                sbuf[k] = e
            rinv = 1.0 / ssum
            for k in range(_K):
                plsc.addupdate_scatter(a_v, [rows, ibuf[k]], sbuf[k] * rinv,
                                       mask=valid)
            return _
        lax.fori_loop(0, _NG, _group, None)

        pltpu.sync_copy(a_v, a_hbm.at[b])


def _sc_attention(S, inxs, radj):
    mesh = plsc.VectorSubcoreMesh(core_axis_name="c", subcore_axis_name="s")
    f = functools.partial(
        pl.kernel,
        mesh=mesh,
        compiler_params=pltpu.CompilerParams(needs_layout_passes=False,
                                             use_tc_tiling_on_sc=False),
        out_type=jax.ShapeDtypeStruct((_B, _T, _T), jnp.float32),
        scratch_types=[
            pltpu.VMEM((_T, _T), jnp.float32),
            pltpu.VMEM((_T, _T), jnp.float32),
            pltpu.VMEM((_T, _K), jnp.int32),
            pltpu.VMEM((_T, _K), jnp.float32),
            pltpu.VMEM((_K, _L), jnp.float32),
            pltpu.VMEM((_K, _L), jnp.int32),
        ],
    )(_sc_body)
    return f(S, inxs, radj)


def _ln(x, g, b, eps=1e-5):
    mu = jnp.mean(x, axis=-1, keepdims=True)
    xc = x - mu
    var = jnp.mean(xc * xc, axis=-1, keepdims=True)
    return xc * lax.rsqrt(var + eps) * g + b


def _s3_body(a_ref, x_ref, wv_ref, wo_ref, g1_ref, be1_ref, w1_ref, bf1_ref,
             w2_ref, bf2_ref, g2_ref, be2_ref, o_ref):
    xb = x_ref[0]
    A = a_ref[0]
    v = jnp.dot(xb, wv_ref[...], preferred_element_type=jnp.float32)
    h = jnp.dot(A, v, preferred_element_type=jnp.float32)
    h = jnp.dot(h, wo_ref[...], preferred_element_type=jnp.float32)
    h = jnp.maximum(h, 0.0)
    y = _ln(xb + h, g1_ref[...], be1_ref[...])
    f = jnp.dot(y, w1_ref[...], preferred_element_type=jnp.float32)
    f = jnp.maximum(f + bf1_ref[...], 0.0)
    f = jnp.dot(f, w2_ref[...], preferred_element_type=jnp.float32)
    f = f + bf2_ref[...]
    o_ref[0] = _ln(y + f, g2_ref[...], be2_ref[...])


def _epilog(A, x, Wv, Wo, g1, be1, W1, bf1, W2, bf2, g2, be2):
    full = lambda shape: pl.BlockSpec(shape, lambda b: (0,) * len(shape))
    return pl.pallas_call(
        _s3_body,
        grid=(_B,),
        in_specs=[
            pl.BlockSpec((1, _T, _T), lambda b: (b, 0, 0)),
            pl.BlockSpec((1, _T, _C), lambda b: (b, 0, 0)),
            full((_C, _C)), full((_C, _C)),
            full((1, _C)), full((1, _C)),
            full((_C, _DFF)), full((1, _DFF)),
            full((_DFF, _C)), full((1, _C)),
            full((1, _C)), full((1, _C)),
        ],
        out_specs=pl.BlockSpec((1, _T, _C), lambda b: (b, 0, 0)),
        out_shape=jax.ShapeDtypeStruct((_B, _T, _C), jnp.float32),
    )(A, x, Wv, Wo, g1.reshape(1, _C), be1.reshape(1, _C), W1,
      bf1.reshape(1, _DFF), W2, bf2.reshape(1, _C), g2.reshape(1, _C),
      be2.reshape(1, _C))


def kernel(x, radj, inxs, Wq, Wk, Wv, Wo, g1, be1, W1, bf1, W2, bf2, g2, be2):
    inxs = inxs.astype(jnp.int32)
    S = _scores(x, Wq, Wk)
    A = _sc_attention(S, inxs, radj)
    return _epilog(A, x, Wv, Wo, g1, be1, W1, bf1, W2, bf2, g2, be2)


# speedup vs baseline: 58.1057x; 2.5414x over previous
import functools
import math

import jax
import jax.numpy as jnp
from jax import lax
from jax.experimental import pallas as pl
from jax.experimental.pallas import tpu as pltpu
from jax.experimental.pallas import tpu_sc as plsc

_B, _T, _C, _K = 64, 200, 128, 16
_DFF = int(_C * 1.5)
_NC, _NS, _L = 2, 16, 16
_NW = _NC * _NS
_BPW = _B // _NW
_NG = (_T + _L - 1) // _L
_NSLAB = 2
_TP = _NSLAB * _C
_G = 8


def _s1_body(x_ref, idx_ref, radj_ref, wqk_ref, s_ref, aux_ref):
    xg = x_ref[...].reshape(_G * _T, _C)
    qk = jnp.dot(xg, wqk_ref[...], preferred_element_type=jnp.float32)
    zpad = jnp.zeros((_TP - _T, _C), jnp.float32)
    for g in range(_G):
        q = qk[g * _T:(g + 1) * _T, :_C]
        k = qk[g * _T:(g + 1) * _T, _C:]
        kpad = jnp.concatenate([k, zpad], axis=0)
        s = lax.dot_general(q, kpad, (((1,), (1,)), ((), ())),
                            preferred_element_type=jnp.float32)
        s_ref[g, :_T] = s[:, :_C]
        s_ref[g, _T:] = s[:, _C:]
    idx_f = lax.bitcast_convert_type(idx_ref[...], jnp.float32)
    aux_ref[...] = jnp.concatenate(
        [radj_ref[...], idx_f,
         jnp.zeros((_G, _T, _C - 2 * _K), jnp.float32)], axis=2)


def _scores(x, inxs, radj, Wqk):
    nb = x.shape[0]
    return pl.pallas_call(
        _s1_body,
        grid=(nb // _G,),
        in_specs=[
            pl.BlockSpec((_G, _T, _C), lambda b: (b, 0, 0)),
            pl.BlockSpec((_G, _T, _K), lambda b: (b, 0, 0)),
            pl.BlockSpec((_G, _T, _K), lambda b: (b, 0, 0)),
            pl.BlockSpec((_C, 2 * _C), lambda b: (0, 0)),
        ],
        out_specs=[
            pl.BlockSpec((_G, _NSLAB * _T, _C), lambda b: (b, 0, 0)),
            pl.BlockSpec((_G, _T, _C), lambda b: (b, 0, 0)),
        ],
        out_shape=[
            jax.ShapeDtypeStruct((nb, _NSLAB * _T, _C), jnp.float32),
            jax.ShapeDtypeStruct((nb, _T, _C), jnp.float32),
        ],
    )(x, inxs, radj, Wqk)


def _sc_body(bpw, s_hbm, aux_hbm, a_hbm, s_v, a_v, aux_v):
    wid = lax.axis_index("s") * _NC + lax.axis_index("c")
    zero16 = jnp.zeros((_L,), jnp.float32)
    lanes = jnp.arange(_L, dtype=jnp.int32)

    for i in range(bpw):
        b = wid * bpw + i
        pltpu.sync_copy(s_hbm.at[b], s_v)
        pltpu.sync_copy(aux_hbm.at[b], aux_v)

        def _zrow(t, carry):
            for j in range(_C // _L):
                a_v[t, pl.ds(j * _L, _L)] = zero16
            return carry
        lax.fori_loop(0, _NSLAB * _T, _zrow, None)

        def _group(g, carry):
            tb = g * _L
            rows_raw = tb + lanes
            valid = rows_raw < _T
            rows = jnp.minimum(rows_raw, _T - 1)
            nks, sks = [], []
            m = jnp.full((_L,), -jnp.inf, jnp.float32)
            for k in range(_K):
                rk = plsc.load_gather(aux_v, [rows, jnp.full((_L,), k, jnp.int32)])
                nf = plsc.load_gather(aux_v, [rows, jnp.full((_L,), _K + k, jnp.int32)])
                nk = plsc.bitcast(nf, jnp.int32)
                srow = (nk >> 7) * _T + rows
                sk = plsc.load_gather(s_v, [srow, nk & 127]) + rk
                m = jnp.maximum(m, sk)
                nks.append(nk)
                sks.append(sk)
            es = [jnp.exp(sk - m) for sk in sks]
            ssum = zero16
            for e in es:
                ssum = ssum + e
            rinv = 1.0 / ssum
            for k in range(_K):
                nk = nks[k]
                srow = (nk >> 7) * _T + rows
                plsc.addupdate_scatter(a_v, [srow, nk & 127], es[k] * rinv,
                                       mask=valid)
            return carry
        lax.fori_loop(0, _NG, _group, None)

        pltpu.sync_copy(a_v, a_hbm.at[b])


def _sc_attention(S2, aux):
    nb = S2.shape[0]
    mesh = plsc.VectorSubcoreMesh(core_axis_name="c", subcore_axis_name="s")
    f = functools.partial(
        pl.kernel,
        mesh=mesh,
        compiler_params=pltpu.CompilerParams(needs_layout_passes=False,
                                             use_tc_tiling_on_sc=False),
        out_type=jax.ShapeDtypeStruct((nb, _NSLAB * _T, _C), jnp.float32),
        scratch_types=[
            pltpu.VMEM((_NSLAB * _T, _C), jnp.float32),
            pltpu.VMEM((_NSLAB * _T, _C), jnp.float32),
            pltpu.VMEM((_T, _C), jnp.float32),
        ],
    )(functools.partial(_sc_body, nb // _NW))
    return f(S2, aux)


def _ln(x, g, b, eps=1e-5):
    mu = jnp.mean(x, axis=-1, keepdims=True)
    xc = x - mu
    var = jnp.mean(xc * xc, axis=-1, keepdims=True)
    return xc * lax.rsqrt(var + eps) * g + b


def _s3_body(a_ref, x_ref, wv_ref, wo_ref, g1_ref, be1_ref, w1_ref, bf1_ref,
             w2_ref, bf2_ref, g2_ref, be2_ref, o_ref):
    xg = x_ref[...].reshape(_G * _T, _C)
    v = jnp.dot(xg, wv_ref[...], preferred_element_type=jnp.float32)
    zpad = jnp.zeros((_TP - _T, _C), jnp.float32)
    hs = []
    for g in range(_G):
        vb = v[g * _T:(g + 1) * _T]
        vhi = jnp.concatenate([vb[_C:], zpad], axis=0)
        hg = jnp.dot(a_ref[g, :_T], vb[:_C], preferred_element_type=jnp.float32)
        hg = hg + jnp.dot(a_ref[g, _T:], vhi, preferred_element_type=jnp.float32)
        hs.append(hg)
    h = jnp.concatenate(hs, axis=0)
    h = jnp.dot(h, wo_ref[...], preferred_element_type=jnp.float32)
    h = jnp.maximum(h, 0.0)
    y = _ln(xg + h, g1_ref[...], be1_ref[...])
    f = jnp.dot(y, w1_ref[...], preferred_element_type=jnp.float32)
    f = jnp.maximum(f + bf1_ref[...], 0.0)
    f = jnp.dot(f, w2_ref[...], preferred_element_type=jnp.float32)
    f = f + bf2_ref[...]
    o_ref[...] = _ln(y + f, g2_ref[...], be2_ref[...]).reshape(_G, _T, _C)


def _epilog(A2, x, Wv, Wo, g1, be1, W1, bf1, W2, bf2, g2, be2):
    nb = x.shape[0]
    full = lambda shape: pl.BlockSpec(shape, lambda b: (0,) * len(shape))
    return pl.pallas_call(
        _s3_body,
        grid=(nb // _G,),
        in_specs=[
            pl.BlockSpec((_G, _NSLAB * _T, _C), lambda b: (b, 0, 0)),
            pl.BlockSpec((_G, _T, _C), lambda b: (b, 0, 0)),
            full((_C, _C)), full((_C, _C)),
            full((1, _C)), full((1, _C)),
            full((_C, _DFF)), full((1, _DFF)),
            full((_DFF, _C)), full((1, _C)),
            full((1, _C)), full((1, _C)),
        ],
        out_specs=pl.BlockSpec((_G, _T, _C), lambda b: (b, 0, 0)),
        out_shape=jax.ShapeDtypeStruct((nb, _T, _C), jnp.float32),
    )(A2, x, Wv, Wo, g1.reshape(1, _C), be1.reshape(1, _C), W1,
      bf1.reshape(1, _DFF), W2, bf2.reshape(1, _C), g2.reshape(1, _C),
      be2.reshape(1, _C))


def kernel(x, radj, inxs, Wq, Wk, Wv, Wo, g1, be1, W1, bf1, W2, bf2, g2, be2):
    inxs = inxs.astype(jnp.int32)
    Wqk = jnp.concatenate([Wq * (1.0 / math.sqrt(_C)), Wk], axis=1)
    nh = _B // 2
    halves = []
    prev_a2 = xprev = None
    for h in range(2):
        xh = lax.slice_in_dim(x, h * nh, (h + 1) * nh, axis=0)
        ih = lax.slice_in_dim(inxs, h * nh, (h + 1) * nh, axis=0)
        rh = lax.slice_in_dim(radj, h * nh, (h + 1) * nh, axis=0)
        S2, aux = _scores(xh, ih, rh, Wqk)
        if prev_a2 is not None:
            prev_a2, S2, aux = lax.optimization_barrier((prev_a2, S2, aux))
            halves.append(
                _epilog(prev_a2, xprev, Wv, Wo, g1, be1, W1, bf1, W2, bf2,
                        g2, be2))
        prev_a2 = _sc_attention(S2, aux)
        xprev = xh
    halves.append(
        _epilog(prev_a2, xprev, Wv, Wo, g1, be1, W1, bf1, W2, bf2, g2, be2))
    return jnp.concatenate(halves, axis=0)


# speedup vs baseline: 59.0921x; 1.0170x over previous
import functools
import math

import jax
import jax.numpy as jnp
from jax import lax
from jax.experimental import pallas as pl
from jax.experimental.pallas import tpu as pltpu
from jax.experimental.pallas import tpu_sc as plsc

_B, _T, _C, _K = 64, 200, 128, 16
_DFF = int(_C * 1.5)
_NC, _NS, _L = 2, 16, 16
_NW = _NC * _NS
_BPW = _B // _NW
_NG = (_T + _L - 1) // _L
_NSLAB = 2
_TP = _NSLAB * _C
_G = 8


def _s1_body(x_ref, idx_ref, radj_ref, wqk_ref, s_ref, aux_ref):
    xg = x_ref[...].reshape(_G * _T, _C)
    qk = jnp.dot(xg, wqk_ref[...], preferred_element_type=jnp.float32)
    zpad = jnp.zeros((_TP - _T, _C), jnp.float32)
    for g in range(_G):
        q = qk[g * _T:(g + 1) * _T, :_C]
        k = qk[g * _T:(g + 1) * _T, _C:]
        kpad = jnp.concatenate([k, zpad], axis=0)
        s = lax.dot_general(q, kpad, (((1,), (1,)), ((), ())),
                            preferred_element_type=jnp.float32)
        s_ref[g, :_T] = s[:, :_C]
        s_ref[g, _T:] = s[:, _C:]
    idx_f = lax.bitcast_convert_type(idx_ref[...], jnp.float32)
    aux_ref[...] = jnp.concatenate(
        [radj_ref[...], idx_f,
         jnp.zeros((_G, _T, _C - 2 * _K), jnp.float32)], axis=2)


def _scores(x, inxs, radj, Wqk, nb, off):
    return pl.pallas_call(
        _s1_body,
        grid=(nb // _G,),
        in_specs=[
            pl.BlockSpec((_G, _T, _C), lambda b, o=off: (b + o, 0, 0)),
            pl.BlockSpec((_G, _T, _K), lambda b, o=off: (b + o, 0, 0)),
            pl.BlockSpec((_G, _T, _K), lambda b, o=off: (b + o, 0, 0)),
            pl.BlockSpec((_C, 2 * _C), lambda b: (0, 0)),
        ],
        out_specs=[
            pl.BlockSpec((_G, _NSLAB * _T, _C), lambda b: (b, 0, 0)),
            pl.BlockSpec((_G, _T, _C), lambda b: (b, 0, 0)),
        ],
        out_shape=[
            jax.ShapeDtypeStruct((nb, _NSLAB * _T, _C), jnp.float32),
            jax.ShapeDtypeStruct((nb, _T, _C), jnp.float32),
        ],
    )(x, inxs, radj, Wqk)


def _sc_body(bpw, s_hbm, aux_hbm, a_hbm, s_v, a_v, aux_v):
    wid = lax.axis_index("s") * _NC + lax.axis_index("c")
    zero16 = jnp.zeros((_L,), jnp.float32)
    lanes = jnp.arange(_L, dtype=jnp.int32)

    for i in range(bpw):
        b = wid * bpw + i
        pltpu.sync_copy(s_hbm.at[b], s_v)
        pltpu.sync_copy(aux_hbm.at[b], aux_v)

        def _zrow(t, carry):
            for j in range(_C // _L):
                a_v[t, pl.ds(j * _L, _L)] = zero16
            return carry
        lax.fori_loop(0, _NSLAB * _T, _zrow, None)

        def _group(g, carry):
            tb = g * _L
            rows_raw = tb + lanes
            valid = rows_raw < _T
            rows = jnp.minimum(rows_raw, _T - 1)
            nks, sks = [], []
            m = jnp.full((_L,), -jnp.inf, jnp.float32)
            for k in range(_K):
                rk = plsc.load_gather(aux_v, [rows, jnp.full((_L,), k, jnp.int32)])
                nf = plsc.load_gather(aux_v, [rows, jnp.full((_L,), _K + k, jnp.int32)])
                nk = plsc.bitcast(nf, jnp.int32)
                srow = (nk >> 7) * _T + rows
                sk = plsc.load_gather(s_v, [srow, nk & 127]) + rk
                m = jnp.maximum(m, sk)
                nks.append(nk)
                sks.append(sk)
            es = [jnp.exp(sk - m) for sk in sks]
            ssum = zero16
            for e in es:
                ssum = ssum + e
            rinv = 1.0 / ssum
            for k in range(_K):
                nk = nks[k]
                srow = (nk >> 7) * _T + rows
                plsc.addupdate_scatter(a_v, [srow, nk & 127], es[k] * rinv,
                                       mask=valid)
            return carry
        lax.fori_loop(0, _NG, _group, None)

        pltpu.sync_copy(a_v, a_hbm.at[b])


def _sc_attention(S2, aux):
    nb = S2.shape[0]
    mesh = plsc.VectorSubcoreMesh(core_axis_name="c", subcore_axis_name="s")
    f = functools.partial(
        pl.kernel,
        mesh=mesh,
        compiler_params=pltpu.CompilerParams(needs_layout_passes=False,
                                             use_tc_tiling_on_sc=False),
        out_type=jax.ShapeDtypeStruct((nb, _NSLAB * _T, _C), jnp.float32),
        scratch_types=[
            pltpu.VMEM((_NSLAB * _T, _C), jnp.float32),
            pltpu.VMEM((_NSLAB * _T, _C), jnp.float32),
            pltpu.VMEM((_T, _C), jnp.float32),
        ],
    )(functools.partial(_sc_body, nb // _NW))
    return f(S2, aux)


def _ln(x, g, b, eps=1e-5):
    mu = jnp.mean(x, axis=-1, keepdims=True)
    xc = x - mu
    var = jnp.mean(xc * xc, axis=-1, keepdims=True)
    return xc * lax.rsqrt(var + eps) * g + b


def _s3_body(a_ref, x_ref, wv_ref, wo_ref, g1_ref, be1_ref, w1_ref, bf1_ref,
             w2_ref, bf2_ref, g2_ref, be2_ref, o_ref):
    xg = x_ref[...].reshape(_G * _T, _C)
    v = jnp.dot(xg, wv_ref[...], preferred_element_type=jnp.float32)
    zpad = jnp.zeros((_TP - _T, _C), jnp.float32)
    hs = []
    for g in range(_G):
        vb = v[g * _T:(g + 1) * _T]
        vhi = jnp.concatenate([vb[_C:], zpad], axis=0)
        hg = jnp.dot(a_ref[g, :_T], vb[:_C], preferred_element_type=jnp.float32)
        hg = hg + jnp.dot(a_ref[g, _T:], vhi, preferred_element_type=jnp.float32)
        hs.append(hg)
    h = jnp.concatenate(hs, axis=0)
    h = jnp.dot(h, wo_ref[...], preferred_element_type=jnp.float32)
    h = jnp.maximum(h, 0.0)
    y = _ln(xg + h, g1_ref[...], be1_ref[...])
    f = jnp.dot(y, w1_ref[...], preferred_element_type=jnp.float32)
    f = jnp.maximum(f + bf1_ref[...], 0.0)
    f = jnp.dot(f, w2_ref[...], preferred_element_type=jnp.float32)
    f = f + bf2_ref[...]
    o_ref[...] = _ln(y + f, g2_ref[...], be2_ref[...]).reshape(_G, _T, _C)


def _epilog(A2, x, Wv, Wo, g1, be1, W1, bf1, W2, bf2, g2, be2, nb, off):
    full = lambda shape: pl.BlockSpec(shape, lambda b: (0,) * len(shape))
    return pl.pallas_call(
        _s3_body,
        grid=(nb // _G,),
        in_specs=[
            pl.BlockSpec((_G, _NSLAB * _T, _C), lambda b: (b, 0, 0)),
            pl.BlockSpec((_G, _T, _C), lambda b, o=off: (b + o, 0, 0)),
            full((_C, _C)), full((_C, _C)),
            full((1, _C)), full((1, _C)),
            full((_C, _DFF)), full((1, _DFF)),
            full((_DFF, _C)), full((1, _C)),
            full((1, _C)), full((1, _C)),
        ],
        out_specs=pl.BlockSpec((_G, _T, _C), lambda b: (b, 0, 0)),
        out_shape=jax.ShapeDtypeStruct((nb, _T, _C), jnp.float32),
    )(A2, x, Wv, Wo, g1.reshape(1, _C), be1.reshape(1, _C), W1,
      bf1.reshape(1, _DFF), W2, bf2.reshape(1, _C), g2.reshape(1, _C),
      be2.reshape(1, _C))


def kernel(x, radj, inxs, Wq, Wk, Wv, Wo, g1, be1, W1, bf1, W2, bf2, g2, be2):
    inxs = inxs.astype(jnp.int32)
    Wqk = jnp.concatenate([Wq * (1.0 / math.sqrt(_C)), Wk], axis=1)
    nh = _B // 2
    halves = []
    prev_a2 = prev_off = None
    for h in range(2):
        off = h * (nh // _G)
        S2, aux = _scores(x, inxs, radj, Wqk, nh, off)
        if prev_a2 is not None:
            prev_a2, S2, aux = lax.optimization_barrier((prev_a2, S2, aux))
            halves.append(
                _epilog(prev_a2, x, Wv, Wo, g1, be1, W1, bf1, W2, bf2,
                        g2, be2, nh, prev_off))
        prev_a2 = _sc_attention(S2, aux)
        prev_off = off
    halves.append(
        _epilog(prev_a2, x, Wv, Wo, g1, be1, W1, bf1, W2, bf2, g2, be2,
                nh, prev_off))
    return jnp.concatenate(halves, axis=0)


# speedup vs baseline: 62.4851x; 1.0574x over previous
import functools
import math

import jax
import jax.numpy as jnp
from jax import lax
from jax.experimental import pallas as pl
from jax.experimental.pallas import tpu as pltpu
from jax.experimental.pallas import tpu_sc as plsc

_B, _T, _C, _K = 64, 200, 128, 16
_DFF = int(_C * 1.5)
_NC, _NS, _L = 2, 16, 16
_NW = _NC * _NS
_BPW = _B // _NW
_NG = (_T + _L - 1) // _L
_NSLAB = 2
_TP = _NSLAB * _C
_G = 8


def _s1_body(x_ref, idx_ref, radj_ref, wqk_ref, s_ref, aux_ref):
    xg = x_ref[...].reshape(_G * _T, _C)
    qk = jnp.dot(xg, wqk_ref[...], preferred_element_type=jnp.float32)
    zpad = jnp.zeros((_TP - _T, _C), jnp.float32)
    for g in range(_G):
        q = qk[g * _T:(g + 1) * _T, :_C]
        k = qk[g * _T:(g + 1) * _T, _C:]
        kpad = jnp.concatenate([k, zpad], axis=0)
        s = lax.dot_general(q, kpad, (((1,), (1,)), ((), ())),
                            preferred_element_type=jnp.float32)
        s_ref[g, :_T] = s[:, :_C]
        s_ref[g, _T:] = s[:, _C:]
    idx_f = lax.bitcast_convert_type(idx_ref[...], jnp.float32)
    aux_ref[...] = jnp.concatenate(
        [radj_ref[...], idx_f,
         jnp.zeros((_G, _T, _C - 2 * _K), jnp.float32)], axis=2)


def _scores(x, inxs, radj, Wqk, nb, off):
    return pl.pallas_call(
        _s1_body,
        grid=(nb // _G,),
        in_specs=[
            pl.BlockSpec((_G, _T, _C), lambda b, o=off: (b + o, 0, 0)),
            pl.BlockSpec((_G, _T, _K), lambda b, o=off: (b + o, 0, 0)),
            pl.BlockSpec((_G, _T, _K), lambda b, o=off: (b + o, 0, 0)),
            pl.BlockSpec((_C, 2 * _C), lambda b: (0, 0)),
        ],
        out_specs=[
            pl.BlockSpec((_G, _NSLAB * _T, _C), lambda b: (b, 0, 0)),
            pl.BlockSpec((_G, _T, _C), lambda b: (b, 0, 0)),
        ],
        out_shape=[
            jax.ShapeDtypeStruct((nb, _NSLAB * _T, _C), jnp.float32),
            jax.ShapeDtypeStruct((nb, _T, _C), jnp.float32),
        ],
    )(x, inxs, radj, Wqk)


def _sc_body(bpw, s_hbm, aux_hbm, a_hbm, s_v, a_v, aux_v, sem_s, sem_x):
    wid = lax.axis_index("s") * _NC + lax.axis_index("c")
    zero16 = jnp.zeros((_L,), jnp.float32)
    lanes = jnp.arange(_L, dtype=jnp.int32)

    for i in range(bpw):
        b = wid * bpw + i
        cp_s = pltpu.async_copy(s_hbm.at[b], s_v, sem_s)
        cp_x = pltpu.async_copy(aux_hbm.at[b], aux_v, sem_x)

        def _zrow(t, carry):
            for j in range(_C // _L):
                a_v[t, pl.ds(j * _L, _L)] = zero16
            return carry
        lax.fori_loop(0, _NSLAB * _T, _zrow, None)
        cp_x.wait()
        cp_s.wait()

        def _group(g, carry):
            tb = g * _L
            rows_raw = tb + lanes
            valid = rows_raw < _T
            rows = jnp.minimum(rows_raw, _T - 1)
            nks, sks = [], []
            m = jnp.full((_L,), -jnp.inf, jnp.float32)
            for k in range(_K):
                rk = plsc.load_gather(aux_v, [rows, jnp.full((_L,), k, jnp.int32)])
                nf = plsc.load_gather(aux_v, [rows, jnp.full((_L,), _K + k, jnp.int32)])
                nk = plsc.bitcast(nf, jnp.int32)
                srow = (nk >> 7) * _T + rows
                sk = plsc.load_gather(s_v, [srow, nk & 127]) + rk
                m = jnp.maximum(m, sk)
                nks.append(nk)
                sks.append(sk)
            es = [jnp.exp(sk - m) for sk in sks]
            ssum = zero16
            for e in es:
                ssum = ssum + e
            rinv = 1.0 / ssum
            for k in range(_K):
                nk = nks[k]
                srow = (nk >> 7) * _T + rows
                plsc.addupdate_scatter(a_v, [srow, nk & 127], es[k] * rinv,
                                       mask=valid)
            return carry
        lax.fori_loop(0, _NG, _group, None)

        pltpu.sync_copy(a_v, a_hbm.at[b])


def _sc_attention(S2, aux):
    nb = S2.shape[0]
    mesh = plsc.VectorSubcoreMesh(core_axis_name="c", subcore_axis_name="s")
    f = functools.partial(
        pl.kernel,
        mesh=mesh,
        compiler_params=pltpu.CompilerParams(needs_layout_passes=False,
                                             use_tc_tiling_on_sc=False),
        out_type=jax.ShapeDtypeStruct((nb, _NSLAB * _T, _C), jnp.float32),
        scratch_types=[
            pltpu.VMEM((_NSLAB * _T, _C), jnp.float32),
            pltpu.VMEM((_NSLAB * _T, _C), jnp.float32),
            pltpu.VMEM((_T, _C), jnp.float32),
            pltpu.SemaphoreType.DMA,
            pltpu.SemaphoreType.DMA,
        ],
    )(functools.partial(_sc_body, nb // _NW))
    return f(S2, aux)


def _ln(x, g, b, eps=1e-5):
    mu = jnp.mean(x, axis=-1, keepdims=True)
    xc = x - mu
    var = jnp.mean(xc * xc, axis=-1, keepdims=True)
    return xc * lax.rsqrt(var + eps) * g + b


def _s3_body(a_ref, x_ref, wv_ref, wo_ref, g1_ref, be1_ref, w1_ref, bf1_ref,
             w2_ref, bf2_ref, g2_ref, be2_ref, o_ref):
    xg = x_ref[...].reshape(_G * _T, _C)
    v = jnp.dot(xg, wv_ref[...], preferred_element_type=jnp.float32)
    zpad = jnp.zeros((_TP - _T, _C), jnp.float32)
    hs = []
    for g in range(_G):
        vb = v[g * _T:(g + 1) * _T]
        vhi = jnp.concatenate([vb[_C:], zpad], axis=0)
        hg = jnp.dot(a_ref[g, :_T], vb[:_C], preferred_element_type=jnp.float32)
        hg = hg + jnp.dot(a_ref[g, _T:], vhi, preferred_element_type=jnp.float32)
        hs.append(hg)
    h = jnp.concatenate(hs, axis=0)
    h = jnp.dot(h, wo_ref[...], preferred_element_type=jnp.float32)
    h = jnp.maximum(h, 0.0)
    y = _ln(xg + h, g1_ref[...], be1_ref[...])
    f = jnp.dot(y, w1_ref[...], preferred_element_type=jnp.float32)
    f = jnp.maximum(f + bf1_ref[...], 0.0)
    f = jnp.dot(f, w2_ref[...], preferred_element_type=jnp.float32)
    f = f + bf2_ref[...]
    o_ref[...] = _ln(y + f, g2_ref[...], be2_ref[...]).reshape(_G, _T, _C)


def _epilog(A2, x, Wv, Wo, g1, be1, W1, bf1, W2, bf2, g2, be2, nb, off):
    full = lambda shape: pl.BlockSpec(shape, lambda b: (0,) * len(shape))
    return pl.pallas_call(
        _s3_body,
        grid=(nb // _G,),
        in_specs=[
            pl.BlockSpec((_G, _NSLAB * _T, _C), lambda b: (b, 0, 0)),
            pl.BlockSpec((_G, _T, _C), lambda b, o=off: (b + o, 0, 0)),
            full((_C, _C)), full((_C, _C)),
            full((1, _C)), full((1, _C)),
            full((_C, _DFF)), full((1, _DFF)),
            full((_DFF, _C)), full((1, _C)),
            full((1, _C)), full((1, _C)),
        ],
        out_specs=pl.BlockSpec((_G, _T, _C), lambda b: (b, 0, 0)),
        out_shape=jax.ShapeDtypeStruct((nb, _T, _C), jnp.float32),
    )(A2, x, Wv, Wo, g1.reshape(1, _C), be1.reshape(1, _C), W1,
      bf1.reshape(1, _DFF), W2, bf2.reshape(1, _C), g2.reshape(1, _C),
      be2.reshape(1, _C))


def kernel(x, radj, inxs, Wq, Wk, Wv, Wo, g1, be1, W1, bf1, W2, bf2, g2, be2):
    inxs = inxs.astype(jnp.int32)
    Wqk = jnp.concatenate([Wq * (1.0 / math.sqrt(_C)), Wk], axis=1)
    nh = _B // 2
    halves = []
    prev_a2 = prev_off = None
    for h in range(2):
        off = h * (nh // _G)
        S2, aux = _scores(x, inxs, radj, Wqk, nh, off)
        if prev_a2 is not None:
            prev_a2, S2, aux = lax.optimization_barrier((prev_a2, S2, aux))
            halves.append(
                _epilog(prev_a2, x, Wv, Wo, g1, be1, W1, bf1, W2, bf2,
                        g2, be2, nh, prev_off))
        prev_a2 = _sc_attention(S2, aux)
        prev_off = off
    halves.append(
        _epilog(prev_a2, x, Wv, Wo, g1, be1, W1, bf1, W2, bf2, g2, be2,
                nh, prev_off))
    return jnp.concatenate(halves, axis=0)
